# baseline (speedup 1.0000x reference)
import functools

import jax
import jax.numpy as jnp
from jax import lax
from jax.experimental import pallas as pl
from jax.experimental.pallas import tpu as pltpu
from jax.experimental.pallas import tpu_sc as plsc

DIM = 128
NUM_CORES = 2
NUM_SUBCORES = 16
NUM_WORKERS = NUM_CORES * NUM_SUBCORES
ROWS_PER_CHUNK = 8
NBUF = 2
PADDED_SEQ = 56


def kernel(input, table):
    batch, seq = input.shape
    num_idx = batch * seq
    idx = input.reshape(num_idx).astype(jnp.int32)

    chunk_idx = ROWS_PER_CHUNK * seq
    rows_per_w = batch // NUM_WORKERS
    n_chunks = rows_per_w // ROWS_PER_CHUNK
    assert rows_per_w * NUM_WORKERS == batch
    assert n_chunks * ROWS_PER_CHUNK == rows_per_w and n_chunks % NBUF == 0
    buf_rows = (ROWS_PER_CHUNK - 1) * seq + PADDED_SEQ

    mesh = plsc.VectorSubcoreMesh(core_axis_name="c", subcore_axis_name="s")

    @functools.partial(
        pl.kernel,
        mesh=mesh,
        out_type=jax.ShapeDtypeStruct((batch * PADDED_SEQ, DIM), jnp.float32),
        scratch_types=(
            [pltpu.VMEM((chunk_idx,), jnp.int32) for _ in range(NBUF)]
            + [pltpu.VMEM((buf_rows, DIM), jnp.float32) for _ in range(NBUF)]
            + [pltpu.SemaphoreType.DMA for _ in range(2 * NBUF)]
        ),
    )
    def gather_kernel(table_hbm, idx_hbm, out_hbm, *scratch):
        idx_v = scratch[:NBUF]
        rows_v = scratch[NBUF:2 * NBUF]
        g_sem = scratch[2 * NBUF:3 * NBUF]
        o_sem = scratch[3 * NBUF:]
        wid = lax.axis_index("s") * NUM_CORES + lax.axis_index("c")
        row0 = wid * rows_per_w

        def start_gather(c, b):
            off = (row0 + c * ROWS_PER_CHUNK) * seq
            pltpu.sync_copy(idx_hbm.at[pl.ds(off, chunk_idx)], idx_v[b])
            pltpu.async_copy(table_hbm.at[idx_v[b]],
                             rows_v[b].at[pl.ds(0, chunk_idx)], g_sem[b])

        def writeback(c, b, fire):
            row = row0 + c * ROWS_PER_CHUNK
            for i in range(ROWS_PER_CHUNK):
                cp = pltpu.make_async_copy(
                    rows_v[b].at[pl.ds(i * seq, PADDED_SEQ)],
                    out_hbm.at[pl.ds((row + i) * PADDED_SEQ, PADDED_SEQ)],
                    o_sem[b])
                if fire:
                    cp.start()
                else:
                    cp.wait()

        def step(c, b, issue_next):
            pltpu.make_async_copy(table_hbm.at[idx_v[b]],
                                  rows_v[b].at[pl.ds(0, chunk_idx)],
                                  g_sem[b]).wait()
            writeback(c, b, fire=True)
            if issue_next:
                writeback(c, b, fire=False)
                start_gather(c + NBUF, b)

        for b in range(NBUF):
            start_gather(b, b)

        @pl.loop(0, n_chunks - NBUF, step=NBUF)
        def _(j):
            for b in range(NBUF):
                step(j + b, b, issue_next=True)

        for b in range(NBUF):
            c = n_chunks - NBUF + b
            step(c, b, issue_next=False)
            writeback(c, b, fire=False)

    out = gather_kernel(table, idx)
    return out.reshape(batch, PADDED_SEQ, DIM)[:, :seq, :]


# speedup vs baseline: 2.0492x; 2.0492x over previous
import functools

import jax
import jax.numpy as jnp
from jax import lax
from jax.experimental import pallas as pl
from jax.experimental.pallas import tpu as pltpu
from jax.experimental.pallas import tpu_sc as plsc

DIM = 128
NUM_CORES = 2
NUM_SUBCORES = 16
NUM_WORKERS = NUM_CORES * NUM_SUBCORES
CHUNK = 400
NBUF = 2


def kernel(input, table):
    batch, seq = input.shape
    num_idx = batch * seq
    idx = input.astype(jnp.int32).T.reshape(num_idx)

    b_per_w = num_idx // NUM_WORKERS
    n_chunks = b_per_w // CHUNK
    assert b_per_w * NUM_WORKERS == num_idx
    assert n_chunks * CHUNK == b_per_w and n_chunks % NBUF == 0

    mesh = plsc.VectorSubcoreMesh(core_axis_name="c", subcore_axis_name="s")

    @functools.partial(
        pl.kernel,
        mesh=mesh,
        out_type=jax.ShapeDtypeStruct((num_idx, DIM), jnp.float32),
        scratch_types=(
            [pltpu.VMEM((CHUNK,), jnp.int32) for _ in range(NBUF)]
            + [pltpu.VMEM((CHUNK, DIM), jnp.float32) for _ in range(NBUF)]
            + [pltpu.SemaphoreType.DMA for _ in range(2 * NBUF)]
        ),
    )
    def gather_kernel(table_hbm, idx_hbm, out_hbm, *scratch):
        idx_v = scratch[:NBUF]
        rows_v = scratch[NBUF:2 * NBUF]
        g_sem = scratch[2 * NBUF:3 * NBUF]
        o_sem = scratch[3 * NBUF:]
        wid = lax.axis_index("s") * NUM_CORES + lax.axis_index("c")
        base = wid * b_per_w

        def start_gather(off, b):
            pltpu.sync_copy(idx_hbm.at[pl.ds(off, CHUNK)], idx_v[b])
            pltpu.async_copy(table_hbm.at[idx_v[b]], rows_v[b], g_sem[b])

        def step(off, b, issue_next):
            pltpu.make_async_copy(table_hbm.at[idx_v[b]], rows_v[b],
                                  g_sem[b]).wait()
            pltpu.async_copy(rows_v[b], out_hbm.at[pl.ds(off, CHUNK)],
                             o_sem[b])
            if issue_next:
                pltpu.make_async_copy(rows_v[b],
                                      out_hbm.at[pl.ds(off, CHUNK)],
                                      o_sem[b]).wait()
                start_gather(off + NBUF * CHUNK, b)

        for b in range(NBUF):
            start_gather(base + b * CHUNK, b)

        @pl.loop(0, n_chunks - NBUF, step=NBUF)
        def _(j):
            for b in range(NBUF):
                step(base + (j + b) * CHUNK, b, issue_next=True)

        for b in range(NBUF):
            off = base + (n_chunks - NBUF + b) * CHUNK
            step(off, b, issue_next=False)
            pltpu.make_async_copy(rows_v[b], out_hbm.at[pl.ds(off, CHUNK)],
                                  o_sem[b]).wait()

    out = gather_kernel(table, idx)
    return out.reshape(seq, batch, DIM).transpose(1, 0, 2)


# speedup vs baseline: 2.0553x; 1.0030x over previous
import functools

import jax
import jax.numpy as jnp
from jax import lax
from jax.experimental import pallas as pl
from jax.experimental.pallas import tpu as pltpu
from jax.experimental.pallas import tpu_sc as plsc

DIM = 128
NUM_CORES = 2
NUM_SUBCORES = 16
NUM_WORKERS = NUM_CORES * NUM_SUBCORES
CHUNK = 400
NBUF = 2


def kernel(input, table):
    batch, seq = input.shape
    num_idx = batch * seq
    idx = input.astype(jnp.int32).T.reshape(num_idx)

    b_per_w = num_idx // NUM_WORKERS
    n_chunks = b_per_w // CHUNK
    assert b_per_w * NUM_WORKERS == num_idx
    assert n_chunks * CHUNK == b_per_w and n_chunks % NBUF == 0

    mesh = plsc.VectorSubcoreMesh(core_axis_name="c", subcore_axis_name="s")

    @functools.partial(
        pl.kernel,
        mesh=mesh,
        out_type=jax.ShapeDtypeStruct((num_idx, DIM), jnp.float32),
        scratch_types=(
            [pltpu.VMEM((CHUNK,), jnp.int32) for _ in range(NBUF)]
            + [pltpu.VMEM((CHUNK, DIM), jnp.float32) for _ in range(NBUF)]
            + [pltpu.SemaphoreType.DMA for _ in range(3 * NBUF)]
        ),
    )
    def gather_kernel(table_hbm, idx_hbm, out_hbm, *scratch):
        idx_v = scratch[:NBUF]
        rows_v = scratch[NBUF:2 * NBUF]
        g_sem = scratch[2 * NBUF:3 * NBUF]
        o_sem = scratch[3 * NBUF:4 * NBUF]
        i_sem = scratch[4 * NBUF:]
        wid = lax.axis_index("s") * NUM_CORES + lax.axis_index("c")
        base = wid * b_per_w

        def idx_copy(off, b):
            return pltpu.make_async_copy(idx_hbm.at[pl.ds(off, CHUNK)],
                                         idx_v[b], i_sem[b])

        def gather_copy(b):
            return pltpu.make_async_copy(table_hbm.at[idx_v[b]], rows_v[b],
                                         g_sem[b])

        def out_copy(off, b):
            return pltpu.make_async_copy(rows_v[b],
                                         out_hbm.at[pl.ds(off, CHUNK)],
                                         o_sem[b])

        def step(off, b, issue_next):
            gather_copy(b).wait()
            if issue_next:
                idx_copy(off + NBUF * CHUNK, b).start()
            out_copy(off, b).start()
            if issue_next:
                out_copy(off, b).wait()
                idx_copy(off + NBUF * CHUNK, b).wait()
                gather_copy(b).start()

        for b in range(NBUF):
            idx_copy(base + b * CHUNK, b).start()
            idx_copy(base + b * CHUNK, b).wait()
            gather_copy(b).start()

        @pl.loop(0, n_chunks - NBUF, step=NBUF)
        def _(j):
            for b in range(NBUF):
                step(base + (j + b) * CHUNK, b, issue_next=True)

        for b in range(NBUF):
            off = base + (n_chunks - NBUF + b) * CHUNK
            step(off, b, issue_next=False)
            out_copy(off, b).wait()

    out = gather_kernel(table, idx)
    return out.reshape(seq, batch, DIM).transpose(1, 0, 2)


# speedup vs baseline: 2.0813x; 1.0126x over previous
import functools

import jax
import jax.numpy as jnp
from jax import lax
from jax.experimental import pallas as pl
from jax.experimental.pallas import tpu as pltpu
from jax.experimental.pallas import tpu_sc as plsc

DIM = 128
NUM_CORES = 2
NUM_SUBCORES = 16
NUM_WORKERS = NUM_CORES * NUM_SUBCORES
CHUNK = 200
NBUF = 4


def kernel(input, table):
    batch, seq = input.shape
    num_idx = batch * seq
    idx = input.astype(jnp.int32).T.reshape(num_idx)

    b_per_w = num_idx // NUM_WORKERS
    n_chunks = b_per_w // CHUNK
    assert b_per_w * NUM_WORKERS == num_idx
    assert n_chunks * CHUNK == b_per_w and n_chunks % NBUF == 0

    mesh = plsc.VectorSubcoreMesh(core_axis_name="c", subcore_axis_name="s")

    @functools.partial(
        pl.kernel,
        mesh=mesh,
        out_type=jax.ShapeDtypeStruct((num_idx, DIM), jnp.float32),
        scratch_types=(
            [pltpu.VMEM((CHUNK,), jnp.int32) for _ in range(NBUF)]
            + [pltpu.VMEM((CHUNK, DIM), jnp.float32) for _ in range(NBUF)]
            + [pltpu.SemaphoreType.DMA for _ in range(3 * NBUF)]
        ),
    )
    def gather_kernel(table_hbm, idx_hbm, out_hbm, *scratch):
        idx_v = scratch[:NBUF]
        rows_v = scratch[NBUF:2 * NBUF]
        g_sem = scratch[2 * NBUF:3 * NBUF]
        o_sem = scratch[3 * NBUF:4 * NBUF]
        i_sem = scratch[4 * NBUF:]
        wid = lax.axis_index("s") * NUM_CORES + lax.axis_index("c")
        base = wid * b_per_w

        def idx_copy(off, b):
            return pltpu.make_async_copy(idx_hbm.at[pl.ds(off, CHUNK)],
                                         idx_v[b], i_sem[b])

        def gather_copy(b):
            return pltpu.make_async_copy(table_hbm.at[idx_v[b]], rows_v[b],
                                         g_sem[b])

        def out_copy(off, b):
            return pltpu.make_async_copy(rows_v[b],
                                         out_hbm.at[pl.ds(off, CHUNK)],
                                         o_sem[b])

        def step(off, b, issue_next):
            gather_copy(b).wait()
            if issue_next:
                idx_copy(off + NBUF * CHUNK, b).start()
            out_copy(off, b).start()
            if issue_next:
                out_copy(off, b).wait()
                idx_copy(off + NBUF * CHUNK, b).wait()
                gather_copy(b).start()

        for b in range(NBUF):
            idx_copy(base + b * CHUNK, b).start()
            idx_copy(base + b * CHUNK, b).wait()
            gather_copy(b).start()

        @pl.loop(0, n_chunks - NBUF, step=NBUF)
        def _(j):
            for b in range(NBUF):
                step(base + (j + b) * CHUNK, b, issue_next=True)

        for b in range(NBUF):
            off = base + (n_chunks - NBUF + b) * CHUNK
            step(off, b, issue_next=False)
            out_copy(off, b).wait()

    out = gather_kernel(table, idx)
    return out.reshape(seq, batch, DIM).transpose(1, 0, 2)
